# baseline (speedup 1.0000x reference)
import functools

import jax
import jax.numpy as jnp
from jax import lax
from jax.experimental import pallas as pl
from jax.experimental.pallas import tpu as pltpu
from jax.experimental.pallas import tpu_sc as plsc

NUM_VIRTUAL_TOKENS = 200
TOKEN_DIM = 128
BATCH = 1024

NC = 2
NS = 16
NW = NC * NS

N_CHUNKS = BATCH // NW
NBUF = 3


@functools.partial(
    pl.kernel,
    out_type=jax.ShapeDtypeStruct(
        (BATCH, NUM_VIRTUAL_TOKENS, TOKEN_DIM), jnp.float32),
    mesh=plsc.VectorSubcoreMesh(
        core_axis_name="c", subcore_axis_name="s", num_cores=NC,
        num_subcores=NS),
    scratch_types=[
        pltpu.VMEM_SHARED((NUM_VIRTUAL_TOKENS, TOKEN_DIM), jnp.float32),
        [pltpu.VMEM((NUM_VIRTUAL_TOKENS,), jnp.int32)
         for _ in range(BATCH // NW)],
        [pltpu.VMEM((NUM_VIRTUAL_TOKENS, TOKEN_DIM), jnp.float32)
         for _ in range(NBUF)],
        pltpu.SemaphoreType.DMA,
        pltpu.SemaphoreType.DMA,
        pltpu.SemaphoreType.DMA,
    ],
)
def _gather_kernel(idx_hbm, table_hbm, out_hbm, w_v, idx_v, rows_v, i_sem,
                   g_sem, s_sem):
    wid = lax.axis_index("s") * NC + lax.axis_index("c")
    row_base = wid * N_CHUNKS

    idx_copies = [
        pltpu.async_copy(idx_hbm.at[row_base + i], idx_v[i], i_sem)
        for i in range(N_CHUNKS)
    ]

    @pl.when(lax.axis_index("s") == 0)
    def _():
        pltpu.sync_copy(table_hbm, w_v)

    plsc.subcore_barrier()

    def start_gather(i):
        idx_copies[i].wait()
        return pltpu.async_copy(w_v.at[idx_v[i]], rows_v[i % NBUF], g_sem)

    gathers = [None] * N_CHUNKS
    scatters = [None] * N_CHUNKS
    for i in range(NBUF - 1):
        gathers[i] = start_gather(i)
    for i in range(N_CHUNKS):
        if i + NBUF - 1 < N_CHUNKS:
            if i >= 1:
                scatters[i - 1].wait()
            gathers[i + NBUF - 1] = start_gather(i + NBUF - 1)
        gathers[i].wait()
        scatters[i] = pltpu.async_copy(
            rows_v[i % NBUF], out_hbm.at[row_base + i], s_sem)
    for i in range(N_CHUNKS - NBUF, N_CHUNKS):
        scatters[i].wait()


def kernel(indices, W):
    return _gather_kernel(indices, W)
